# Initial kernel scaffold; baseline (speedup 1.0000x reference)
import jax
import jax.numpy as jnp
from jax.experimental import pallas as pl


def kernel(input_ids, embed_tokens):
    raise NotImplementedError("write your pallas kernel here")


# speedup vs baseline: 1.8572x; 1.8572x over previous
import functools

import jax
import jax.numpy as jnp
from jax import lax
from jax.experimental import pallas as pl
from jax.experimental.pallas import tpu as pltpu
from jax.experimental.pallas import tpu_sc as plsc

VOCAB = 100000
D_MODEL = 1024
BATCH = 4
SEQ = 4096

NUM_CORES = 2
NUM_SUBCORES = 16
NUM_WORKERS = NUM_CORES * NUM_SUBCORES
ROWS = BATCH * SEQ
ROWS_PER_WORKER = ROWS // NUM_WORKERS
WORKERS_PER_B = SEQ // ROWS_PER_WORKER
CHUNK = 32
NCHUNK = ROWS_PER_WORKER // CHUNK

_mesh = plsc.VectorSubcoreMesh(core_axis_name="c", subcore_axis_name="s")


@functools.partial(
    pl.kernel,
    mesh=_mesh,
    out_type=jax.ShapeDtypeStruct((SEQ, BATCH, D_MODEL), jnp.float32),
    scratch_types=[
        pltpu.VMEM((ROWS_PER_WORKER,), jnp.int32),
        pltpu.VMEM((CHUNK, D_MODEL), jnp.float32),
        pltpu.SemaphoreType.DMA,
    ],
)
def _embed_gather(ids_hbm, table_hbm, out_hbm, idx_v, buf, gsem):
    wid = lax.axis_index("s") * NUM_CORES + lax.axis_index("c")
    b = wid // WORKERS_PER_B
    s0 = (wid % WORKERS_PER_B) * ROWS_PER_WORKER
    pltpu.sync_copy(ids_hbm.at[pl.ds(wid * ROWS_PER_WORKER, ROWS_PER_WORKER)],
                    idx_v)

    def chunk_body(g, carry):
        off = pl.multiple_of(g * CHUNK, CHUNK)
        pltpu.async_copy(
            table_hbm.at[idx_v.at[pl.ds(off, CHUNK)]], buf, gsem
        ).wait()
        pltpu.sync_copy(buf, out_hbm.at[pl.ds(s0 + off, CHUNK), b])
        return carry

    lax.fori_loop(0, NCHUNK, chunk_body, 0)


def kernel(input_ids, embed_tokens):
    ids_flat = input_ids.reshape(-1).astype(jnp.int32)
    hidden = _embed_gather(ids_flat, embed_tokens)
    return hidden, input_ids


# speedup vs baseline: 2.1376x; 1.1510x over previous
import functools

import jax
import jax.numpy as jnp
from jax import lax
from jax.experimental import pallas as pl
from jax.experimental.pallas import tpu as pltpu
from jax.experimental.pallas import tpu_sc as plsc

VOCAB = 100000
D_MODEL = 1024
BATCH = 4
SEQ = 4096

NUM_CORES = 2
NUM_SUBCORES = 16
NUM_WORKERS = NUM_CORES * NUM_SUBCORES
ROWS = BATCH * SEQ
ROWS_PER_WORKER = ROWS // NUM_WORKERS
WORKERS_PER_B = SEQ // ROWS_PER_WORKER
CHUNK = 32
NCHUNK = ROWS_PER_WORKER // CHUNK

_mesh = plsc.VectorSubcoreMesh(core_axis_name="c", subcore_axis_name="s")


@functools.partial(
    pl.kernel,
    mesh=_mesh,
    out_type=jax.ShapeDtypeStruct((SEQ, BATCH, D_MODEL), jnp.float32),
    scratch_types=[
        pltpu.VMEM((ROWS_PER_WORKER,), jnp.int32),
        pltpu.VMEM((CHUNK, D_MODEL), jnp.float32),
        pltpu.VMEM((CHUNK, D_MODEL), jnp.float32),
        pltpu.SemaphoreType.DMA,
        pltpu.SemaphoreType.DMA,
    ],
)
def _embed_gather(ids_hbm, table_hbm, out_hbm, idx_v, buf0, buf1, sem0, sem1):
    wid = lax.axis_index("s") * NUM_CORES + lax.axis_index("c")
    b = wid // WORKERS_PER_B
    s0 = (wid % WORKERS_PER_B) * ROWS_PER_WORKER
    pltpu.sync_copy(ids_hbm.at[pl.ds(wid * ROWS_PER_WORKER, ROWS_PER_WORKER)],
                    idx_v)

    bufs = (buf0, buf1)
    sems = (sem0, sem1)

    def fire(g, buf, sem):
        off = pl.multiple_of(g * CHUNK, CHUNK)
        pltpu.async_copy(table_hbm.at[idx_v.at[pl.ds(off, CHUNK)]], buf, sem)

    fire(0, buf0, sem0)
    fire(1, buf1, sem1)

    def loop_body(h, carry):
        for slot in range(2):
            g = h * 2 + slot
            buf, sem = bufs[slot], sems[slot]
            pltpu.make_async_copy(table_hbm.at[pl.ds(0, CHUNK)], buf,
                                  sem).wait()
            off = pl.multiple_of(g * CHUNK, CHUNK)
            pltpu.sync_copy(buf, out_hbm.at[pl.ds(s0 + off, CHUNK), b])

            @pl.when(g + 2 < NCHUNK)
            def _():
                fire(g + 2, buf, sem)
        return carry

    lax.fori_loop(0, NCHUNK // 2, loop_body, 0)


def kernel(input_ids, embed_tokens):
    ids_flat = input_ids.reshape(-1).astype(jnp.int32)
    hidden = _embed_gather(ids_flat, embed_tokens)
    return hidden, input_ids


# speedup vs baseline: 2.1530x; 1.0072x over previous
import functools

import jax
import jax.numpy as jnp
from jax import lax
from jax.experimental import pallas as pl
from jax.experimental.pallas import tpu as pltpu
from jax.experimental.pallas import tpu_sc as plsc

VOCAB = 100000
D_MODEL = 1024
BATCH = 4
SEQ = 4096

NUM_CORES = 2
NUM_SUBCORES = 16
NUM_WORKERS = NUM_CORES * NUM_SUBCORES
ROWS = BATCH * SEQ
ROWS_PER_WORKER = ROWS // NUM_WORKERS
WORKERS_PER_B = SEQ // ROWS_PER_WORKER
CHUNK = 16
NCHUNK = ROWS_PER_WORKER // CHUNK
NBUF = 4

_mesh = plsc.VectorSubcoreMesh(core_axis_name="c", subcore_axis_name="s")


@functools.partial(
    pl.kernel,
    mesh=_mesh,
    out_type=jax.ShapeDtypeStruct((SEQ, BATCH, D_MODEL), jnp.float32),
    scratch_types=(
        [pltpu.VMEM((ROWS_PER_WORKER,), jnp.int32)]
        + [pltpu.VMEM((CHUNK, D_MODEL), jnp.float32) for _ in range(NBUF)]
        + [pltpu.SemaphoreType.DMA for _ in range(2 * NBUF)]
    ),
)
def _embed_gather(ids_hbm, table_hbm, out_hbm, idx_v, *rest):
    bufs = rest[:NBUF]
    gsems = rest[NBUF:2 * NBUF]
    osems = rest[2 * NBUF:]
    wid = lax.axis_index("s") * NUM_CORES + lax.axis_index("c")
    b = wid // WORKERS_PER_B
    s0 = (wid % WORKERS_PER_B) * ROWS_PER_WORKER
    pltpu.sync_copy(ids_hbm.at[pl.ds(wid * ROWS_PER_WORKER, ROWS_PER_WORKER)],
                    idx_v)

    def fire_gather(g, m):
        off = pl.multiple_of(g * CHUNK, CHUNK)
        pltpu.async_copy(table_hbm.at[idx_v.at[pl.ds(off, CHUNK)]],
                         bufs[m], gsems[m])

    def out_slice(g):
        off = pl.multiple_of(g * CHUNK, CHUNK)
        return out_hbm.at[pl.ds(s0 + off, CHUNK), b]

    fire_gather(0, 0)
    fire_gather(1, 1)

    def loop_body(h, carry):
        for m in range(NBUF):
            g = h * NBUF + m
            pltpu.make_async_copy(table_hbm.at[pl.ds(0, CHUNK)], bufs[m],
                                  gsems[m]).wait()
            pltpu.async_copy(bufs[m], out_slice(g), osems[m])

            m2 = (m + 2) % NBUF

            @pl.when(g >= 2)
            def _():
                pltpu.make_async_copy(bufs[m2], out_slice(g - 2),
                                      osems[m2]).wait()

            @pl.when(g + 2 < NCHUNK)
            def _():
                fire_gather(g + 2, m2)
        return carry

    lax.fori_loop(0, NCHUNK // NBUF, loop_body, 0)

    for g in (NCHUNK - 2, NCHUNK - 1):
        m = g % NBUF
        pltpu.make_async_copy(bufs[m], out_slice(g), osems[m]).wait()


def kernel(input_ids, embed_tokens):
    ids_flat = input_ids.reshape(-1).astype(jnp.int32)
    hidden = _embed_gather(ids_flat, embed_tokens)
    return hidden, input_ids


# speedup vs baseline: 2.1717x; 1.0087x over previous
import functools

import jax
import jax.numpy as jnp
from jax import lax
from jax.experimental import pallas as pl
from jax.experimental.pallas import tpu as pltpu
from jax.experimental.pallas import tpu_sc as plsc

VOCAB = 100000
D_MODEL = 1024
BATCH = 4
SEQ = 4096

NUM_CORES = 2
NUM_SUBCORES = 16
NUM_WORKERS = NUM_CORES * NUM_SUBCORES
ROWS = BATCH * SEQ
ROWS_PER_WORKER = ROWS // NUM_WORKERS
WORKERS_PER_B = SEQ // ROWS_PER_WORKER
CHUNK = 8
NCHUNK = ROWS_PER_WORKER // CHUNK
NBUF = 8
LEAD = NBUF // 2

_mesh = plsc.VectorSubcoreMesh(core_axis_name="c", subcore_axis_name="s")


@functools.partial(
    pl.kernel,
    mesh=_mesh,
    out_type=jax.ShapeDtypeStruct((SEQ, BATCH, D_MODEL), jnp.float32),
    scratch_types=(
        [pltpu.VMEM((ROWS_PER_WORKER,), jnp.int32)]
        + [pltpu.VMEM((CHUNK, D_MODEL), jnp.float32) for _ in range(NBUF)]
        + [pltpu.SemaphoreType.DMA for _ in range(2 * NBUF)]
    ),
)
def _embed_gather(ids_hbm, table_hbm, out_hbm, idx_v, *rest):
    bufs = rest[:NBUF]
    gsems = rest[NBUF:2 * NBUF]
    osems = rest[2 * NBUF:]
    wid = lax.axis_index("s") * NUM_CORES + lax.axis_index("c")
    b = wid // WORKERS_PER_B
    s0 = (wid % WORKERS_PER_B) * ROWS_PER_WORKER
    pltpu.sync_copy(ids_hbm.at[pl.ds(wid * ROWS_PER_WORKER, ROWS_PER_WORKER)],
                    idx_v)

    def fire_gather(g, m):
        off = pl.multiple_of(g * CHUNK, CHUNK)
        pltpu.async_copy(table_hbm.at[idx_v.at[pl.ds(off, CHUNK)]],
                         bufs[m], gsems[m])

    def out_slice(g):
        off = pl.multiple_of(g * CHUNK, CHUNK)
        return out_hbm.at[pl.ds(s0 + off, CHUNK), b]

    for g0 in range(LEAD):
        fire_gather(g0, g0)

    def loop_body(h, carry):
        for m in range(NBUF):
            g = h * NBUF + m
            pltpu.make_async_copy(table_hbm.at[pl.ds(0, CHUNK)], bufs[m],
                                  gsems[m]).wait()
            pltpu.async_copy(bufs[m], out_slice(g), osems[m])

            m2 = (m + LEAD) % NBUF

            @pl.when(g >= LEAD)
            def _():
                pltpu.make_async_copy(bufs[m2], out_slice(g - LEAD),
                                      osems[m2]).wait()

            @pl.when(g + LEAD < NCHUNK)
            def _():
                fire_gather(g + LEAD, m2)
        return carry

    lax.fori_loop(0, NCHUNK // NBUF, loop_body, 0)

    for g in range(NCHUNK - LEAD, NCHUNK):
        m = g % NBUF
        pltpu.make_async_copy(bufs[m], out_slice(g), osems[m]).wait()


def kernel(input_ids, embed_tokens):
    ids_flat = input_ids.reshape(-1).astype(jnp.int32)
    hidden = _embed_gather(ids_flat, embed_tokens)
    return hidden, input_ids
